# baseline (speedup 1.0000x reference)
import functools

import jax
import jax.numpy as jnp
from jax import lax
from jax.experimental import pallas as pl
from jax.experimental.pallas import tpu as pltpu
from jax.experimental.pallas import tpu_sc as plsc

N = 4096
MM_DIM = 256
ST_DIM = 32
COMPS = 5
INV_TAU = 2.0
TOPK = 6
BLK = 512
NBLK = N // BLK


def _encode_body(mm_ref, st_ref, w1_ref, b1_ref, w2_ref, b2_ref, g_ref,
                 be_ref, enc_ref, encn_ref, stn_ref):
    x = mm_ref[...]
    w1 = w1_ref[0]
    h = lax.dot_general(x, w1, (((1,), (1,)), ((), ())),
                        preferred_element_type=jnp.float32)
    h = h + b1_ref[0]
    h = 0.5 * h * (1.0 + lax.erf(h * 0.7071067811865476))
    w2 = w2_ref[0]
    h = lax.dot_general(h, w2, (((1,), (1,)), ((), ())),
                        preferred_element_type=jnp.float32)
    h = h + b2_ref[0]
    mu = jnp.mean(h, axis=0, keepdims=True)
    d = h - mu
    var = jnp.mean(d * d, axis=0, keepdims=True)
    enc = d * jax.lax.rsqrt(var + 1e-5) * g_ref[0] + be_ref[0]
    enc_ref[0] = enc
    nrm = jnp.sqrt(jnp.sum(enc * enc, axis=1, keepdims=True)) + 1e-12
    encn_ref[0] = enc / nrm
    st = st_ref[0]
    snrm = jnp.sqrt(jnp.sum(st * st, axis=1, keepdims=True)) + 1e-12
    stn_ref[0] = st / snrm


def _pass1_body(stn_ref, encn_ref, rowsum_ref, idx_ref):
    c = pl.program_id(0)
    s = stn_ref[0]
    e = encn_ref[0]
    sim = lax.dot_general(s, e, (((1,), (1,)), ((), ())),
                          preferred_element_type=jnp.float32)
    es = jnp.exp(sim * INV_TAU)
    ones = jnp.ones((N, 1), jnp.float32)
    rowsum_ref[0, 0] = lax.dot_general(
        es, ones, (((1,), (0,)), ((), ())),
        preferred_element_type=jnp.float32)[:, 0]
    idx_ref[0, 0] = jnp.argmax(sim, axis=1).astype(jnp.int32) + c * N


def _pass2_body(g_ref, stn_ref, encn_ref, rowsum_ref, loss_ref):
    c = pl.program_id(0)
    b = pl.program_id(1)
    g = g_ref[0]
    s = stn_ref[0]
    e = encn_ref[0]
    mm = lax.dot_general(g, e, (((1,), (1,)), ((), ())),
                         preferred_element_type=jnp.float32)
    sim = lax.dot_general(s, e, (((1,), (1,)), ((), ())),
                          preferred_element_type=jnp.float32)
    v = jnp.max(mm, axis=1, keepdims=True)
    for _ in range(TOPK - 1):
        v = jnp.max(jnp.where(mm < v, mm, -3.0), axis=1, keepdims=True)
    ones = jnp.ones((N, 1), jnp.float32)
    pos = lax.dot_general(
        jnp.where(mm >= v, jnp.exp(sim * INV_TAU), 0.0), ones,
        (((1,), (0,)), ((), ())),
        preferred_element_type=jnp.float32)[:, 0]
    rs = rowsum_ref[0, 0]
    part = jnp.sum(jnp.log(rs) - jnp.log(pos)).reshape(1, 1)

    @pl.when(jnp.logical_and(c == 0, b == 0))
    def _init():
        loss_ref[...] = jnp.zeros((1, 1), jnp.float32)

    loss_ref[...] += part


def _sc_gather(table, idx):
    info = plsc.get_sparse_core_info()
    nw = info.num_cores * info.num_subcores
    rows = table.shape[0]
    per_w = rows // nw
    mesh = plsc.VectorSubcoreMesh(core_axis_name="c", subcore_axis_name="s")

    @functools.partial(
        pl.kernel, mesh=mesh,
        compiler_params=pltpu.CompilerParams(use_tc_tiling_on_sc=False),
        out_type=jax.ShapeDtypeStruct((rows, ST_DIM), jnp.float32),
        scratch_types=[
            pltpu.VMEM((per_w,), jnp.int32),
            pltpu.VMEM((per_w, ST_DIM), jnp.float32),
            pltpu.SemaphoreType.DMA,
        ],
    )
    def gather_k(table_hbm, idx_hbm, out_hbm, idx_v, rows_v, sem):
        wid = lax.axis_index("s") * info.num_cores + lax.axis_index("c")
        base = wid * per_w
        pltpu.sync_copy(idx_hbm.at[pl.ds(base, per_w)], idx_v)
        pltpu.async_copy(table_hbm.at[idx_v], rows_v, sem).wait()
        pltpu.sync_copy(rows_v, out_hbm.at[pl.ds(base, per_w)])

    return gather_k(table, idx)


def kernel(st_feats, mm_feats, W1, b1, W2, b2, gamma, beta):
    st_t = jnp.transpose(st_feats, (2, 0, 1))
    b1r = b1.reshape(COMPS, 1, ST_DIM)
    b2r = b2.reshape(COMPS, 1, ST_DIM)
    gr = gamma.reshape(COMPS, 1, ST_DIM)
    ber = beta.reshape(COMPS, 1, ST_DIM)

    full3 = pl.BlockSpec((1, N, ST_DIM), lambda c: (c, 0, 0))
    small3 = pl.BlockSpec((1, 1, ST_DIM), lambda c: (c, 0, 0))
    enc_all, encn_all, stn_all = pl.pallas_call(
        _encode_body,
        grid=(COMPS,),
        in_specs=[
            pl.BlockSpec((N, MM_DIM), lambda c: (0, 0)),
            full3,
            pl.BlockSpec((1, ST_DIM, MM_DIM), lambda c: (c, 0, 0)),
            small3,
            pl.BlockSpec((1, ST_DIM, ST_DIM), lambda c: (c, 0, 0)),
            small3, small3, small3,
        ],
        out_specs=[full3, full3, full3],
        out_shape=[jax.ShapeDtypeStruct((COMPS, N, ST_DIM), jnp.float32)] * 3,
    )(mm_feats, st_t, W1, b1r, W2, b2r, gr, ber)

    blk_in = pl.BlockSpec((1, BLK, ST_DIM), lambda c, b: (c, b, 0))
    encn_spec = pl.BlockSpec((1, N, ST_DIM), lambda c, b: (c, 0, 0))
    flat_out = pl.BlockSpec((1, 1, BLK), lambda c, b: (c * NBLK + b, 0, 0))
    rowsum, flatidx = pl.pallas_call(
        _pass1_body,
        grid=(COMPS, NBLK),
        in_specs=[blk_in, encn_spec],
        out_specs=[flat_out, flat_out],
        out_shape=[
            jax.ShapeDtypeStruct((COMPS * NBLK, 1, BLK), jnp.float32),
            jax.ShapeDtypeStruct((COMPS * NBLK, 1, BLK), jnp.int32),
        ],
    )(stn_all, encn_all)

    table = encn_all.reshape(COMPS * N, ST_DIM)
    g_flat = _sc_gather(table, flatidx.reshape(COMPS * N))
    g_all = g_flat.reshape(COMPS, N, ST_DIM)

    loss_sum = pl.pallas_call(
        _pass2_body,
        grid=(COMPS, NBLK),
        in_specs=[
            blk_in, blk_in, encn_spec,
            pl.BlockSpec((1, 1, BLK), lambda c, b: (c * NBLK + b, 0, 0)),
        ],
        out_specs=pl.BlockSpec((1, 1), lambda c, b: (0, 0)),
        out_shape=jax.ShapeDtypeStruct((1, 1), jnp.float32),
    )(g_all, stn_all, encn_all, rowsum)

    mm_out = jnp.transpose(enc_all, (1, 2, 0))
    loss = loss_sum[0, 0] / (COMPS * N)
    return (mm_out, loss)


# speedup vs baseline: 1.1026x; 1.1026x over previous
import functools

import jax
import jax.numpy as jnp
from jax import lax
from jax.experimental import pallas as pl
from jax.experimental.pallas import tpu as pltpu
from jax.experimental.pallas import tpu_sc as plsc

N = 4096
MM_DIM = 256
ST_DIM = 32
COMPS = 5
INV_TAU = 2.0
TOPK = 6
BLK = 1024
NBLK = N // BLK


def _encode_body(mm_ref, st_ref, w1_ref, b1_ref, w2_ref, b2_ref, g_ref,
                 be_ref, enc_ref, encn_ref, stn_ref):
    x = mm_ref[...]
    w1 = w1_ref[0]
    h = lax.dot_general(x, w1, (((1,), (1,)), ((), ())),
                        preferred_element_type=jnp.float32)
    h = h + b1_ref[0]
    h = 0.5 * h * (1.0 + lax.erf(h * 0.7071067811865476))
    w2 = w2_ref[0]
    h = lax.dot_general(h, w2, (((1,), (1,)), ((), ())),
                        preferred_element_type=jnp.float32)
    h = h + b2_ref[0]
    mu = jnp.mean(h, axis=0, keepdims=True)
    d = h - mu
    var = jnp.mean(d * d, axis=0, keepdims=True)
    enc = d * jax.lax.rsqrt(var + 1e-5) * g_ref[0] + be_ref[0]
    enc_ref[0] = enc
    nrm = jnp.sqrt(jnp.sum(enc * enc, axis=1, keepdims=True)) + 1e-12
    encn_ref[0] = enc / nrm
    st = st_ref[0]
    snrm = jnp.sqrt(jnp.sum(st * st, axis=1, keepdims=True)) + 1e-12
    stn_ref[0] = st / snrm


def _pass1_body(stn_ref, encn_ref, rowsum_ref, idx_ref):
    c = pl.program_id(0)
    s = stn_ref[0]
    e = encn_ref[0]
    sim = lax.dot_general(s, e, (((1,), (1,)), ((), ())),
                          preferred_element_type=jnp.float32)
    es = jnp.exp(sim * INV_TAU)
    rowsum_ref[0, 0] = jnp.sum(es, axis=1)
    idx_ref[0, 0] = jnp.argmax(sim, axis=1).astype(jnp.int32) + c * N


def _pass2_body(g_ref, stn_ref, encn_ref, rowsum_ref, loss_ref):
    c = pl.program_id(0)
    b = pl.program_id(1)
    g = g_ref[0]
    s = stn_ref[0]
    e = encn_ref[0]
    mm = lax.dot_general(g, e, (((1,), (1,)), ((), ())),
                         preferred_element_type=jnp.float32)
    sim = lax.dot_general(s, e, (((1,), (1,)), ((), ())),
                          preferred_element_type=jnp.float32)
    v = jnp.max(mm, axis=1, keepdims=True)
    for _ in range(TOPK - 1):
        v = jnp.max(jnp.where(mm < v, mm, -3.0), axis=1, keepdims=True)
    pos = jnp.sum(jnp.where(mm >= v, jnp.exp(sim * INV_TAU), 0.0), axis=1)
    rs = rowsum_ref[0, 0]
    part = jnp.sum(jnp.log(rs) - jnp.log(pos)).reshape(1, 1)

    @pl.when(jnp.logical_and(c == 0, b == 0))
    def _init():
        loss_ref[...] = jnp.zeros((1, 1), jnp.float32)

    loss_ref[...] += part


def _sc_gather(table, idx):
    info = plsc.get_sparse_core_info()
    nw = info.num_cores * info.num_subcores
    rows = table.shape[0]
    per_w = rows // nw
    mesh = plsc.VectorSubcoreMesh(core_axis_name="c", subcore_axis_name="s")

    @functools.partial(
        pl.kernel, mesh=mesh,
        compiler_params=pltpu.CompilerParams(use_tc_tiling_on_sc=False),
        out_type=jax.ShapeDtypeStruct((rows, ST_DIM), jnp.float32),
        scratch_types=[
            pltpu.VMEM((per_w,), jnp.int32),
            pltpu.VMEM((per_w, ST_DIM), jnp.float32),
            pltpu.SemaphoreType.DMA,
        ],
    )
    def gather_k(table_hbm, idx_hbm, out_hbm, idx_v, rows_v, sem):
        wid = lax.axis_index("s") * info.num_cores + lax.axis_index("c")
        base = wid * per_w
        pltpu.sync_copy(idx_hbm.at[pl.ds(base, per_w)], idx_v)
        pltpu.async_copy(table_hbm.at[idx_v], rows_v, sem).wait()
        pltpu.sync_copy(rows_v, out_hbm.at[pl.ds(base, per_w)])

    return gather_k(table, idx)


def kernel(st_feats, mm_feats, W1, b1, W2, b2, gamma, beta):
    st_t = jnp.transpose(st_feats, (2, 0, 1))
    b1r = b1.reshape(COMPS, 1, ST_DIM)
    b2r = b2.reshape(COMPS, 1, ST_DIM)
    gr = gamma.reshape(COMPS, 1, ST_DIM)
    ber = beta.reshape(COMPS, 1, ST_DIM)

    full3 = pl.BlockSpec((1, N, ST_DIM), lambda c: (c, 0, 0))
    small3 = pl.BlockSpec((1, 1, ST_DIM), lambda c: (c, 0, 0))
    enc_all, encn_all, stn_all = pl.pallas_call(
        _encode_body,
        grid=(COMPS,),
        in_specs=[
            pl.BlockSpec((N, MM_DIM), lambda c: (0, 0)),
            full3,
            pl.BlockSpec((1, ST_DIM, MM_DIM), lambda c: (c, 0, 0)),
            small3,
            pl.BlockSpec((1, ST_DIM, ST_DIM), lambda c: (c, 0, 0)),
            small3, small3, small3,
        ],
        out_specs=[full3, full3, full3],
        out_shape=[jax.ShapeDtypeStruct((COMPS, N, ST_DIM), jnp.float32)] * 3,
    )(mm_feats, st_t, W1, b1r, W2, b2r, gr, ber)

    blk_in = pl.BlockSpec((1, BLK, ST_DIM), lambda c, b: (c, b, 0))
    encn_spec = pl.BlockSpec((1, N, ST_DIM), lambda c, b: (c, 0, 0))
    flat_out = pl.BlockSpec((1, 1, BLK), lambda c, b: (c * NBLK + b, 0, 0))
    rowsum, flatidx = pl.pallas_call(
        _pass1_body,
        grid=(COMPS, NBLK),
        in_specs=[blk_in, encn_spec],
        out_specs=[flat_out, flat_out],
        out_shape=[
            jax.ShapeDtypeStruct((COMPS * NBLK, 1, BLK), jnp.float32),
            jax.ShapeDtypeStruct((COMPS * NBLK, 1, BLK), jnp.int32),
        ],
    )(stn_all, encn_all)

    table = encn_all.reshape(COMPS * N, ST_DIM)
    g_flat = _sc_gather(table, flatidx.reshape(COMPS * N))
    g_all = g_flat.reshape(COMPS, N, ST_DIM)

    loss_sum = pl.pallas_call(
        _pass2_body,
        grid=(COMPS, NBLK),
        in_specs=[
            blk_in, blk_in, encn_spec,
            pl.BlockSpec((1, 1, BLK), lambda c, b: (c * NBLK + b, 0, 0)),
        ],
        out_specs=pl.BlockSpec((1, 1), lambda c, b: (0, 0)),
        out_shape=jax.ShapeDtypeStruct((1, 1), jnp.float32),
    )(g_all, stn_all, encn_all, rowsum)

    mm_out = jnp.transpose(enc_all, (1, 2, 0))
    loss = loss_sum[0, 0] / (COMPS * N)
    return (mm_out, loss)
